# baseline (speedup 1.0000x reference)
import functools

import jax
import jax.numpy as jnp
from jax import lax
from jax.experimental import pallas as pl
from jax.experimental.pallas import tpu as pltpu
from jax.experimental.pallas import tpu_sc as plsc

B = 16384
NF = 26
VOCAB = 100000
E = 16
ND = 100
NCONT = ND - NF
OUT = NF * E + NCONT

NC = 2
NS = 16
NW = NC * NS
BPW = B // NW
R = 128
NCHUNK = BPW // R


@functools.partial(
    pl.kernel,
    mesh=plsc.VectorSubcoreMesh(core_axis_name="c", subcore_axis_name="s"),
    out_type=jax.ShapeDtypeStruct((B, OUT), jnp.float32),
    compiler_params=pltpu.CompilerParams(
        use_tc_tiling_on_sc=False, needs_layout_passes=False
    ),
    scratch_types=[
        pltpu.VMEM((R // 2 * ND,), jnp.int32),
        pltpu.VMEM((NF * R,), jnp.int32),
        pltpu.VMEM((NF * R, E), jnp.float32),
        pltpu.VMEM((R, OUT), jnp.float32),
        pltpu.SemaphoreType.DMA,
        pltpu.SemaphoreType.DMA,
    ],
)
def _sc_embed(x_hbm, w_hbm, out_hbm, x_v, idx_v, emb_v, out_v, sem_in, sem_g):
    wid = lax.axis_index("s") * NC + lax.axis_index("c")
    iota = lax.iota(jnp.int32, 16)
    colstep = iota * ND

    H = R // 2

    for c in range(NCHUNK):
        base = wid * BPW + c * R

        for h in range(2):
            pltpu.async_copy(
                x_hbm.at[pl.ds((base + h * H) * ND, H * ND)], x_v, sem_in
            ).wait()

            for f in range(NF):
                for i in range(H // 16):
                    codes = plsc.load_gather(x_v, [colstep + (i * 16 * ND + f)])
                    idx_v[pl.ds(f * R + h * H + i * 16, 16)] = codes + f * VOCAB

            def cont_body(i, _):
                e = i * 16 + iota
                r = e // NCONT
                j = e - NCONT * r
                src = NF + e + (ND - NCONT) * r
                vals = plsc.load_gather(x_v, [src]).astype(jnp.float32)
                plsc.store_scatter(out_v, [h * H + r, (NF * E) + j], vals)
                return 0

            lax.fori_loop(0, H * NCONT // 16, cont_body, 0)

        gps = [
            pltpu.async_copy(
                w_hbm.at[idx_v.at[pl.ds(f * R, R)]],
                emb_v.at[pl.ds(f * R, R)],
                sem_g,
            )
            for f in range(NF)
        ]
        for gp in gps:
            gp.wait()

        def emb_body(i, _):
            f = i // R
            r = i - f * R
            plsc.store_scatter(out_v, [iota * 0 + r, f * E + iota], emb_v[i, :])
            return 0

        lax.fori_loop(0, NF * R, emb_body, 0)

        pltpu.sync_copy(out_v, out_hbm.at[pl.ds(base, R)])


def kernel(x, W):
    xf = x.reshape(-1)
    wf = W.reshape(NF * VOCAB, E)
    return _sc_embed(xf, wf)


# speedup vs baseline: 1.5598x; 1.5598x over previous
import functools

import jax
import jax.numpy as jnp
from jax import lax
from jax.experimental import pallas as pl
from jax.experimental.pallas import tpu as pltpu
from jax.experimental.pallas import tpu_sc as plsc

B = 16384
NF = 26
VOCAB = 100000
E = 16
ND = 100
NCONT = ND - NF
OUT = NF * E + NCONT
EMB = NF * E
Q = 104

NC = 2
NS = 16
NW = NC * NS
BPW = B // NW
R = 128
NCHUNK = BPW // R


@functools.partial(
    pl.kernel,
    mesh=plsc.VectorSubcoreMesh(core_axis_name="c", subcore_axis_name="s"),
    out_type=jax.ShapeDtypeStruct((B, OUT), jnp.float32),
    compiler_params=pltpu.CompilerParams(
        use_tc_tiling_on_sc=False, needs_layout_passes=False
    ),
    scratch_types=[
        pltpu.VMEM((R // 2 * ND,), jnp.int32),
        pltpu.VMEM((R * EMB,), jnp.int32),
        pltpu.VMEM((R, OUT), jnp.float32),
        pltpu.SemaphoreType.DMA,
        pltpu.SemaphoreType.DMA,
    ],
)
def _sc_embed(x_hbm, w_hbm, out_hbm, x_v, idx_v, out_v, sem_in, sem_g):
    wid = lax.axis_index("s") * NC + lax.axis_index("c")
    iota = lax.iota(jnp.int32, 16)
    evec = iota * VOCAB

    H = R // 2

    for c in range(NCHUNK):
        base = wid * BPW + c * R

        for h in range(2):
            pltpu.async_copy(
                x_hbm.at[pl.ds((base + h * H) * ND, H * ND)], x_v, sem_in
            ).wait()

            def idx_body(r, _):
                ro = (h * H + r) * EMB
                row = r * ND
                for f in range(NF):
                    v16 = x_v[pl.ds(row + f, 16)]
                    idx_v[pl.ds(ro + f * E, 16)] = evec + (v16[0] + f * VOCAB * E)
                return 0

            lax.fori_loop(0, H, idx_body, 0)

            def cont_body(i, _):
                e = i * 16 + iota
                r = e // NCONT
                j = e - NCONT * r
                src = NF + e + (ND - NCONT) * r
                vals = plsc.load_gather(x_v, [src]).astype(jnp.float32)
                plsc.store_scatter(out_v, [h * H + r, EMB + j], vals)
                return 0

            lax.fori_loop(0, H * NCONT // 16, cont_body, 0)

        def fire_body(r, _):
            for q in range(EMB // Q):
                pltpu.async_copy(
                    w_hbm.at[idx_v.at[pl.ds(r * EMB + q * Q, Q)]],
                    out_v.at[r, pl.ds(q * Q, Q)],
                    sem_g,
                )
            return 0

        lax.fori_loop(0, R, fire_body, 0)

        def drain_body(r, _):
            for q in range(EMB // Q):
                pltpu.make_async_copy(
                    w_hbm.at[idx_v.at[pl.ds(r * EMB + q * Q, Q)]],
                    out_v.at[r, pl.ds(q * Q, Q)],
                    sem_g,
                ).wait()
            return 0

        lax.fori_loop(0, R, drain_body, 0)

        pltpu.sync_copy(out_v, out_hbm.at[pl.ds(base, R)])


def kernel(x, W):
    xf = x.reshape(-1)
    wt = W.transpose(0, 2, 1).reshape(-1)
    return _sc_embed(xf, wt)
